# baseline (speedup 1.0000x reference)
import functools

import jax
import jax.numpy as jnp
from jax import lax
from jax.experimental import pallas as pl
from jax.experimental.pallas import tpu as pltpu
from jax.experimental.pallas import tpu_sc as plsc

B = 4096
L = 50
DIM = 128
F = 100
VOCAB = 100000

_NC = 2
_NS = 16
_NW = _NC * _NS
_EPW = B // _NW

_mesh = plsc.VectorSubcoreMesh(core_axis_name="c", subcore_axis_name="s")


@functools.partial(
    pl.kernel,
    mesh=_mesh,
    out_type=jax.ShapeDtypeStruct((B, DIM), jnp.float32),
    scratch_types=[
        pltpu.VMEM((L, _EPW), jnp.int32),
        pltpu.VMEM((_EPW, DIM), jnp.float32),
        pltpu.SemaphoreType.DMA,
    ],
)
def _pool_sum(xT_hbm, embed_hbm, out_hbm, idx_v, acc_v, sem):
    wid = lax.axis_index("s") * _NC + lax.axis_index("c")
    base = wid * _EPW
    pltpu.sync_copy(xT_hbm.at[:, pl.ds(base, _EPW)], idx_v)
    pltpu.async_copy(embed_hbm.at[idx_v.at[0]], acc_v, sem).wait()
    copies = [
        pltpu.async_copy(embed_hbm.at[idx_v.at[l]], acc_v, sem, add=True)
        for l in range(1, L)
    ]
    for cp in copies:
        cp.wait()
    pltpu.sync_copy(acc_v, out_hbm.at[pl.ds(base, _EPW)])


def _h_body(msum_ref, w1_ref, b1_ref, out_ref):
    out_ref[...] = (
        jnp.dot(msum_ref[...], w1_ref[...], preferred_element_type=jnp.float32)
        * (1.0 / L)
        + b1_ref[...]
    )


_VT = 512
_NVFULL = VOCAB // _VT
_VREM = VOCAB - _NVFULL * _VT
_NBUF = 4


def _out_body(h_ref, w2_ref, b2_ref, out_hbm, b0, b1_, b2_, b3, sems):
    i = pl.program_id(0)
    slot = lax.rem(i, _NBUF)
    bufs = (b0, b1_, b2_, b3)

    val = (
        jnp.dot(h_ref[...], w2_ref[...], preferred_element_type=jnp.float32)
        + b2_ref[...]
    )

    for k in range(_NBUF):
        @pl.when(slot == k)
        def _site(k=k):
            @pl.when(i >= _NBUF)
            def _drain():
                pltpu.make_async_copy(
                    bufs[k], out_hbm.at[:, pl.ds((i - _NBUF) * _VT, _VT)],
                    sems.at[k],
                ).wait()

            bufs[k][...] = val
            pltpu.make_async_copy(
                bufs[k], out_hbm.at[:, pl.ds(i * _VT, _VT)], sems.at[k]
            ).start()

    @pl.when(i == _NVFULL - 1)
    def _drain_all():
        for k in range(_NBUF):
            j = i - (_NBUF - 1) + k
            s = lax.rem(j, _NBUF)
            for m in range(_NBUF):
                @pl.when(s == m)
                def _dw(m=m, j=j):
                    pltpu.make_async_copy(
                        bufs[m], out_hbm.at[:, pl.ds(j * _VT, _VT)],
                        sems.at[m],
                    ).wait()


def _tail_body(h_ref, w2t_ref, b2t_ref, _aliased_ref, out_ref):
    out_ref[...] = (
        jnp.dot(h_ref[...], w2t_ref[...], preferred_element_type=jnp.float32)
        + b2t_ref[...]
    )


def _simple_out_body(h_ref, w2_ref, b2_ref, out_ref):
    out_ref[...] = (
        jnp.dot(h_ref[...], w2_ref[...], preferred_element_type=jnp.float32)
        + b2_ref[...]
    )


def kernel(x, embed, W1, b1, W2, b2):
    xT = jnp.transpose(x).astype(jnp.int32)
    msum = _pool_sum(xT, embed)

    h = pl.pallas_call(
        _h_body,
        out_shape=jax.ShapeDtypeStruct((B, F), jnp.float32),
    )(msum, W1, b1.reshape(1, F))

    b2r = b2.reshape(1, VOCAB)
    nv = pl.cdiv(VOCAB, 2048)
    out = pl.pallas_call(
        _simple_out_body,
        grid=(nv, B // 1024),
        in_specs=[
            pl.BlockSpec((1024, F), lambda v, b: (b, 0)),
            pl.BlockSpec((F, 2048), lambda v, b: (0, v)),
            pl.BlockSpec((1, 2048), lambda v, b: (0, v)),
        ],
        out_specs=pl.BlockSpec((1024, 2048), lambda v, b: (b, v)),
        out_shape=jax.ShapeDtypeStruct((B, VOCAB), jnp.float32),
    )(h, W2, b2r)
    return out


# speedup vs baseline: 3.4118x; 3.4118x over previous
import functools

import jax
import jax.numpy as jnp
from jax import lax
from jax.experimental import pallas as pl
from jax.experimental.pallas import tpu as pltpu
from jax.experimental.pallas import tpu_sc as plsc

B = 4096
L = 50
DIM = 128
F = 100
VOCAB = 100000

_NC = 2
_NS = 16
_NW = _NC * _NS
_EPW = B // _NW

_mesh = plsc.VectorSubcoreMesh(core_axis_name="c", subcore_axis_name="s")


@functools.partial(
    pl.kernel,
    mesh=_mesh,
    out_type=jax.ShapeDtypeStruct((B, DIM), jnp.float32),
    scratch_types=[
        pltpu.VMEM((L, _EPW), jnp.int32),
        pltpu.VMEM((_EPW, DIM), jnp.float32),
        pltpu.SemaphoreType.DMA,
    ],
)
def _pool_sum(xT_hbm, embed_hbm, out_hbm, idx_v, acc_v, sem):
    wid = lax.axis_index("s") * _NC + lax.axis_index("c")
    base = wid * _EPW
    pltpu.sync_copy(xT_hbm.at[:, pl.ds(base, _EPW)], idx_v)
    pltpu.async_copy(embed_hbm.at[idx_v.at[0]], acc_v, sem).wait()
    copies = [
        pltpu.async_copy(embed_hbm.at[idx_v.at[l]], acc_v, sem, add=True)
        for l in range(1, L)
    ]
    for cp in copies:
        cp.wait()
    pltpu.sync_copy(acc_v, out_hbm.at[pl.ds(base, _EPW)])


def _h_body(msum_ref, w1_ref, b1_ref, out_ref):
    out_ref[...] = (
        jnp.dot(msum_ref[...], w1_ref[...], preferred_element_type=jnp.float32)
        * (1.0 / L)
        + b1_ref[...]
    )


_VT = 1024


def _outT_body(w2_ref, h_ref, b2_ref, out_ref):
    acc = lax.dot_general(
        w2_ref[...], h_ref[...],
        (((0,), (1,)), ((), ())),
        preferred_element_type=jnp.float32,
    )
    out_ref[...] = acc + b2_ref[...]


def kernel(x, embed, W1, b1, W2, b2):
    xT = jnp.transpose(x).astype(jnp.int32)
    msum = _pool_sum(xT, embed)

    h = pl.pallas_call(
        _h_body,
        out_shape=jax.ShapeDtypeStruct((B, F), jnp.float32),
    )(msum, W1, b1.reshape(1, F))

    outT = pl.pallas_call(
        _outT_body,
        grid=(pl.cdiv(VOCAB, _VT),),
        in_specs=[
            pl.BlockSpec((F, _VT), lambda v: (0, v)),
            pl.BlockSpec((B, F), lambda v: (0, 0)),
            pl.BlockSpec((_VT, 1), lambda v: (v, 0)),
        ],
        out_specs=pl.BlockSpec((_VT, B), lambda v: (v, 0)),
        out_shape=jax.ShapeDtypeStruct((VOCAB, B), jnp.float32),
    )(W2, h, b2.reshape(VOCAB, 1))
    return jnp.transpose(outT)
